# baseline (speedup 1.0000x reference)
import functools

import jax
import jax.numpy as jnp
from jax import lax
from jax.experimental import pallas as pl
from jax.experimental.pallas import tpu as pltpu
from jax.experimental.pallas import tpu_sc as plsc

_B = 16384
_D = 32
_NC = 2
_NS = 16
_NW = _NC * _NS
_BPW = _B // _NW
_CH = 128


_mesh = plsc.VectorSubcoreMesh(core_axis_name="c", subcore_axis_name="s")


@functools.partial(
    pl.kernel,
    mesh=_mesh,
    out_type=(
        jax.ShapeDtypeStruct((_B,), jnp.float32),
        jax.ShapeDtypeStruct((_B, _D), jnp.float32),
    ),
    scratch_types=[
        pltpu.VMEM((_BPW,), jnp.int32),
        pltpu.VMEM((_BPW, _D), jnp.float32),
        pltpu.VMEM((_BPW,), jnp.float32),
        pltpu.SemaphoreType.DMA,
        pltpu.SemaphoreType.DMA,
    ],
    compiler_params=pltpu.CompilerParams(use_tc_tiling_on_sc=False),
)
def _emb_lookup(idx_hbm, vect_hbm, bias_hbm, out_b, out_v,
                idx_v, rows_v, bv_v, sem_v, sem_b):
    wid = lax.axis_index("s") * _NC + lax.axis_index("c")
    base = wid * _BPW
    pltpu.sync_copy(idx_hbm.at[pl.ds(base, _BPW)], idx_v)
    copies = []
    for c in range(_BPW // _CH):
        off = c * _CH
        sl = pl.ds(off, _CH)
        copies.append(
            pltpu.async_copy(vect_hbm.at[idx_v.at[sl]], rows_v.at[sl], sem_v))
        copies.append(
            pltpu.async_copy(bias_hbm.at[idx_v.at[sl]], bv_v.at[sl], sem_b))
    for cp in copies:
        cp.wait()
    pltpu.sync_copy(rows_v, out_v.at[pl.ds(base, _BPW)])
    pltpu.sync_copy(bv_v, out_b.at[pl.ds(base, _BPW)])


def kernel(index, vect, bias):
    idx = index.astype(jnp.int32)
    bias_flat = bias.reshape(-1)
    out_b, out_v = _emb_lookup(idx, vect, bias_flat)
    return (out_b, out_v)
